# baseline (speedup 1.0000x reference)
import functools

import jax
import jax.numpy as jnp
from jax import lax
from jax.experimental import pallas as pl
from jax.experimental.pallas import tpu as pltpu
from jax.experimental.pallas import tpu_sc as plsc

N = 10000
E = 320000
NUM_MASK = 3000

NC = 2
NS = 16
NW = NC * NS
NPAD = 10240
EW = 10240
EPAD = NW * EW
BLK = 128
NBLK = EW // BLK
RPS = NPAD // NS

_mesh = plsc.VectorSubcoreMesh(core_axis_name="c", subcore_axis_name="s")


def _fill(ref, nrows, width, value):
    v = jnp.full((16,), value, jnp.float32)

    def row(i, _):
        for j in range(width // 16):
            ref[i, pl.ds(j * 16, 16)] = v
        return 0

    lax.fori_loop(0, nrows, row, 0)


@functools.partial(
    pl.kernel,
    out_type=jax.ShapeDtypeStruct((NC, NPAD, 128), jnp.float32),
    mesh=_mesh,
    scratch_types=[
        pltpu.VMEM((NBLK, BLK), jnp.int32),
        pltpu.VMEM((BLK, 128), jnp.float32),
        pltpu.VMEM((64, 128), jnp.float32),
        pltpu.VMEM_SHARED((NPAD, 128), jnp.float32),
        pltpu.SemaphoreType.DMA,
    ],
)
def _deg_sc(dst_hbm, out_hbm, dst_v, ones_v, zb_v, acc, sem):
    c = lax.axis_index("c")
    s = lax.axis_index("s")
    wid = s * NC + c
    _fill(ones_v, BLK, 128, 1.0)
    _fill(zb_v, 64, 128, 0.0)

    def zero(t, _):
        pltpu.sync_copy(zb_v, acc.at[pl.ds(s * RPS + t * 64, 64)])
        return 0

    lax.fori_loop(0, RPS // 64, zero, 0)
    pltpu.sync_copy(dst_hbm.at[wid], dst_v)
    plsc.subcore_barrier()

    def blk(b, _):
        pltpu.async_copy(ones_v, acc.at[dst_v.at[b]], sem, add=True)
        return 0

    lax.fori_loop(0, NBLK, blk, 0)

    def drain(b, _):
        pltpu.make_async_copy(ones_v, acc.at[dst_v.at[0]], sem).wait()
        return 0

    lax.fori_loop(0, NBLK, drain, 0)
    plsc.subcore_barrier()
    pltpu.sync_copy(acc.at[pl.ds(s * RPS, RPS)],
                    out_hbm.at[c, pl.ds(s * RPS, RPS)])


def _make_prop(D):
    HB = NBLK // 2

    @functools.partial(
        pl.kernel,
        out_type=jax.ShapeDtypeStruct((NC, NPAD, D), jnp.float32),
        mesh=_mesh,
        scratch_types=[
            pltpu.VMEM((HB, BLK), jnp.int32),
            pltpu.VMEM((HB, BLK), jnp.int32),
            pltpu.VMEM((BLK, D), jnp.float32),
            pltpu.VMEM((BLK, D), jnp.float32),
            pltpu.VMEM_SHARED((NPAD, D), jnp.float32),
            pltpu.SemaphoreType.DMA,
            pltpu.SemaphoreType.DMA,
            pltpu.SemaphoreType.DMA,
            pltpu.SemaphoreType.DMA,
        ],
    )
    def prop(table_hbm, src_hbm, dst_hbm, out_hbm, src_v, dst_v, rows_a,
             rows_b, acc, sem_ga, sem_gb, sem_sa, sem_sb):
        c = lax.axis_index("c")
        s = lax.axis_index("s")
        wid = s * NC + c
        _fill(rows_a, BLK, D, 0.0)

        def zero(t, _):
            pltpu.sync_copy(rows_a, acc.at[pl.ds(s * RPS + t * BLK, BLK)])
            return 0

        lax.fori_loop(0, RPS // BLK, zero, 0)
        plsc.subcore_barrier()

        def wait_gather(idx_row, rows, sem):
            pltpu.make_async_copy(table_hbm.at[src_v.at[idx_row]], rows,
                                  sem).wait()

        def wait_scatter(idx_row, rows, sem):
            pltpu.make_async_copy(rows, acc.at[dst_v.at[idx_row]], sem).wait()

        for h in range(2):
            pltpu.sync_copy(src_hbm.at[wid, pl.ds(h * HB, HB)], src_v)
            pltpu.sync_copy(dst_hbm.at[wid, pl.ds(h * HB, HB)], dst_v)
            pltpu.async_copy(table_hbm.at[src_v.at[0]], rows_a, sem_ga)
            pltpu.async_copy(table_hbm.at[src_v.at[1]], rows_b, sem_gb)

            def pair(g, _):
                b0 = 2 * g
                wait_gather(b0, rows_a, sem_ga)
                pltpu.async_copy(rows_a, acc.at[dst_v.at[b0]], sem_sa,
                                 add=True)
                wait_gather(b0 + 1, rows_b, sem_gb)
                pltpu.async_copy(rows_b, acc.at[dst_v.at[b0 + 1]], sem_sb,
                                 add=True)

                @pl.when(b0 + 2 < HB)
                def _():
                    wait_scatter(b0, rows_a, sem_sa)
                    pltpu.async_copy(table_hbm.at[src_v.at[b0 + 2]], rows_a,
                                     sem_ga)

                @pl.when(b0 + 3 < HB)
                def _():
                    wait_scatter(b0 + 1, rows_b, sem_sb)
                    pltpu.async_copy(table_hbm.at[src_v.at[b0 + 3]], rows_b,
                                     sem_gb)

                return 0

            lax.fori_loop(0, HB // 2, pair, 0)
            wait_scatter(HB - 2, rows_a, sem_sa)
            wait_scatter(HB - 1, rows_b, sem_sb)
        plsc.subcore_barrier()
        pltpu.sync_copy(acc.at[pl.ds(s * RPS, RPS)],
                        out_hbm.at[c, pl.ds(s * RPS, RPS)])

    return prop


_prop128 = _make_prop(128)

RB = 1000
GRID = N // RB


def _tc_spec(d):
    return pl.BlockSpec((RB, d), lambda i: (i, 0))


def _full_spec(shape):
    return pl.BlockSpec(shape, lambda i: tuple(0 for _ in shape))


def _tc_stage1(deg0, deg1, x, m, pos, W1):
    def body(d0_r, d1_r, x_r, m_r, pos_r, w_r, dis_o, t1_o):
        deg = d0_r[..., 0] + d1_r[..., 0] + 1.0
        dis = (1.0 / jnp.sqrt(deg))[:, None]
        xb = x_r[...] + m_r[...] * pos_r[...]
        t1 = jnp.dot(xb, w_r[...], preferred_element_type=jnp.float32)
        dis_o[...] = dis
        t1_o[...] = t1 * dis

    return pl.pallas_call(
        body,
        grid=(GRID,),
        in_specs=[_tc_spec(128), _tc_spec(128), _tc_spec(128), _tc_spec(1),
                  _full_spec((1, 128)), _full_spec((128, 128))],
        out_specs=[_tc_spec(1), _tc_spec(128)],
        out_shape=[jax.ShapeDtypeStruct((N, 1), jnp.float32),
                   jax.ShapeDtypeStruct((N, 128), jnp.float32)],
    )(deg0, deg1, x, m, pos, W1)


def _tc_stage2(p0, p1, t1, dis, b1, a_enc, W2):
    def body(p0_r, p1_r, t1_r, dis_r, b_r, a_r, w_r, t2_o):
        a = a_r[0, 0]
        dis = dis_r[...]
        tot = dis * (p0_r[...] + p1_r[...] + t1_r[...]) + b_r[...]
        h = jnp.where(tot >= 0, tot, a * tot)
        t2 = dis * jnp.dot(h, w_r[...], preferred_element_type=jnp.float32)
        t2_o[...] = jnp.concatenate(
            [t2, jnp.zeros((t2.shape[0], 64), jnp.float32)], axis=1)

    return pl.pallas_call(
        body,
        grid=(GRID,),
        in_specs=[_tc_spec(128), _tc_spec(128), _tc_spec(128), _tc_spec(1),
                  _full_spec((1, 128)), _full_spec((1, 1)),
                  _full_spec((128, 64))],
        out_specs=_tc_spec(128),
        out_shape=jax.ShapeDtypeStruct((N, 128), jnp.float32),
    )(p0, p1, t1, dis, b1, a_enc, W2)


def _tc_stage3(p0, p1, t2, dis, b2, a_enc, We2d, Wd, m):
    def body(p0_r, p1_r, t2_r, dis_r, b_r, a_r, we_r, wd_r, m_r, t3_o):
        a = a_r[0, 0]
        dis = dis_r[...]
        tot = dis * (p0_r[...] + p1_r[...] + t2_r[...])[:, :64] + b_r[...]
        h2 = jnp.where(tot >= 0, tot, a * tot)
        rec = lax.dot_general(h2, we_r[...], (((1,), (1,)), ((), ())),
                              preferred_element_type=jnp.float32)
        rec = rec * (1.0 - m_r[...])
        t3_o[...] = dis * jnp.dot(rec, wd_r[...],
                                  preferred_element_type=jnp.float32)

    return pl.pallas_call(
        body,
        grid=(GRID,),
        in_specs=[_tc_spec(128), _tc_spec(128), _tc_spec(128), _tc_spec(1),
                  _full_spec((1, 64)), _full_spec((1, 1)),
                  _full_spec((64, 64)), _full_spec((64, 128)), _tc_spec(1)],
        out_specs=_tc_spec(128),
        out_shape=jax.ShapeDtypeStruct((N, 128), jnp.float32),
    )(p0, p1, t2, dis, b2, a_enc, We2d, Wd, m)


def _tc_stage4(p0, p1, t3, dis, bd, a_dec, x, m):
    def body(p0_r, p1_r, t3_r, dis_r, b_r, a_r, x_r, m_r, out_o):
        i = pl.program_id(0)
        a = a_r[0, 0]
        dis = dis_r[...]
        tot = dis * (p0_r[...] + p1_r[...] + t3_r[...]) + b_r[...]
        dec = jnp.where(tot >= 0, tot, a * tot)
        xv = x_r[...]
        xn = xv / jnp.maximum(
            jnp.sqrt(jnp.sum(xv * xv, axis=-1, keepdims=True)), 1e-12)
        rn = dec / jnp.maximum(
            jnp.sqrt(jnp.sum(dec * dec, axis=-1, keepdims=True)), 1e-12)
        cos = jnp.sum(xn * rn, axis=-1)
        e = 1.0 - cos
        contrib = jnp.sum(m_r[..., 0] * e * e) * (1.0 / NUM_MASK)

        @pl.when(i == 0)
        def _():
            out_o[...] = jnp.zeros((1, 1), jnp.float32)

        out_o[...] += jnp.reshape(contrib, (1, 1))

    return pl.pallas_call(
        body,
        grid=(GRID,),
        in_specs=[_tc_spec(128), _tc_spec(128), _tc_spec(128), _tc_spec(1),
                  _full_spec((1, 128)), _full_spec((1, 1)), _tc_spec(128),
                  _tc_spec(1)],
        out_specs=_full_spec((1, 1)),
        out_shape=jax.ShapeDtypeStruct((1, 1), jnp.float32),
    )(p0, p1, t3, dis, bd, a_dec, x, m)


def kernel(x, edge_index, W1, b1, W2, b2, a_enc, W_e2d, Wd, bd, a_dec,
           pos_token):
    f32 = jnp.float32
    ei = edge_index.astype(jnp.int32)
    npadlen = EPAD - E
    pad_i = jnp.arange(npadlen, dtype=jnp.int32)
    src_pad = jnp.concatenate([ei[0], (pad_i * 131) % N]).reshape(NW, NBLK, BLK)
    dst_pad = jnp.concatenate([ei[1], N + pad_i % (NPAD - N)]).reshape(
        NW, NBLK, BLK)

    perm = jax.random.permutation(jax.random.key(42), N)
    mask_nodes = perm[:NUM_MASK]
    m = jnp.zeros((N,), f32).at[mask_nodes].set(1.0).reshape(N, 1)

    degp = _deg_sc(dst_pad)
    dis, t1 = _tc_stage1(degp[0], degp[1], x, m, pos_token, W1)

    part1 = _prop128(t1, src_pad, dst_pad)
    t2 = _tc_stage2(part1[0], part1[1], t1, dis, b1.reshape(1, 128),
                    a_enc.reshape(1, 1), W2)

    part2 = _prop128(t2, src_pad, dst_pad)
    t3 = _tc_stage3(part2[0], part2[1], t2, dis, b2.reshape(1, 64),
                    a_enc.reshape(1, 1), W_e2d, Wd, m)

    part3 = _prop128(t3, src_pad, dst_pad)
    out = _tc_stage4(part3[0], part3[1], t3, dis, bd.reshape(1, 128),
                     a_dec.reshape(1, 1), x, m)
    return out[0, 0]


# speedup vs baseline: 1.2138x; 1.2138x over previous
import functools

import jax
import jax.numpy as jnp
from jax import lax
from jax.experimental import pallas as pl
from jax.experimental.pallas import tpu as pltpu
from jax.experimental.pallas import tpu_sc as plsc

N = 10000
E = 320000
NUM_MASK = 3000

NC = 2
NS = 16
NW = NC * NS
NPAD = 10240
EW = 10240
EPAD = NW * EW
BLK = 128
NBLK = EW // BLK
RPS = NPAD // NS

_mesh = plsc.VectorSubcoreMesh(core_axis_name="c", subcore_axis_name="s")


def _fill(ref, nrows, width, value):
    v = jnp.full((16,), value, jnp.float32)

    def row(i, _):
        for j in range(width // 16):
            ref[i, pl.ds(j * 16, 16)] = v
        return 0

    lax.fori_loop(0, nrows, row, 0)


@functools.partial(
    pl.kernel,
    out_type=jax.ShapeDtypeStruct((NC, NPAD, 128), jnp.float32),
    mesh=_mesh,
    scratch_types=[
        pltpu.VMEM((NBLK, BLK), jnp.int32),
        pltpu.VMEM((BLK, 128), jnp.float32),
        pltpu.VMEM((64, 128), jnp.float32),
        pltpu.VMEM_SHARED((NPAD, 128), jnp.float32),
        pltpu.SemaphoreType.DMA,
    ],
)
def _deg_sc(dst_hbm, out_hbm, dst_v, ones_v, zb_v, acc, sem):
    c = lax.axis_index("c")
    s = lax.axis_index("s")
    wid = s * NC + c
    _fill(ones_v, BLK, 128, 1.0)
    _fill(zb_v, 64, 128, 0.0)

    def zero(t, _):
        pltpu.sync_copy(zb_v, acc.at[pl.ds(s * RPS + t * 64, 64)])
        return 0

    lax.fori_loop(0, RPS // 64, zero, 0)
    pltpu.sync_copy(dst_hbm.at[wid], dst_v)
    plsc.subcore_barrier()

    def blk(b, _):
        pltpu.async_copy(ones_v, acc.at[dst_v.at[b]], sem, add=True)
        return 0

    lax.fori_loop(0, NBLK, blk, 0)

    def drain(b, _):
        pltpu.make_async_copy(ones_v, acc.at[dst_v.at[0]], sem).wait()
        return 0

    lax.fori_loop(0, NBLK, drain, 0)
    plsc.subcore_barrier()
    pltpu.sync_copy(acc.at[pl.ds(s * RPS, RPS)],
                    out_hbm.at[c, pl.ds(s * RPS, RPS)])


def _make_prop(D):
    HB = NBLK // 2

    @functools.partial(
        pl.kernel,
        out_type=jax.ShapeDtypeStruct((NC, NPAD, D), jnp.float32),
        mesh=_mesh,
        scratch_types=[
            pltpu.VMEM((HB, BLK), jnp.int32),
            pltpu.VMEM((HB, BLK), jnp.int32),
            pltpu.VMEM((BLK, D), jnp.float32),
            pltpu.VMEM((BLK, D), jnp.float32),
            pltpu.VMEM_SHARED((NPAD, D), jnp.float32),
            pltpu.SemaphoreType.DMA,
            pltpu.SemaphoreType.DMA,
        ],
    )
    def prop(table_hbm, src_hbm, dst_hbm, out_hbm, src_v, dst_v, rows_a,
             rows_b, acc, sem_ga, sem_gb):
        c = lax.axis_index("c")
        s = lax.axis_index("s")
        wid = s * NC + c
        _fill(rows_a, BLK, D, 0.0)

        def zero(t, _):
            pltpu.sync_copy(rows_a, acc.at[pl.ds(s * RPS + t * BLK, BLK)])
            return 0

        lax.fori_loop(0, RPS // BLK, zero, 0)
        plsc.subcore_barrier()

        for h in range(2):
            pltpu.sync_copy(src_hbm.at[wid, pl.ds(h * HB, HB)], src_v)
            pltpu.sync_copy(dst_hbm.at[wid, pl.ds(h * HB, HB)], dst_v)
            pltpu.async_copy(table_hbm.at[src_v.at[0]], rows_a, sem_ga)

            def pair(g, _):
                b0 = 2 * g
                pltpu.async_copy(table_hbm.at[src_v.at[b0 + 1]], rows_b,
                                 sem_gb)
                pltpu.make_async_copy(table_hbm.at[src_v.at[b0]], rows_a,
                                      sem_ga).wait()
                pltpu.sync_copy(rows_a, acc.at[dst_v.at[b0]], add=True)

                @pl.when(b0 + 2 < HB)
                def _():
                    pltpu.async_copy(table_hbm.at[src_v.at[b0 + 2]], rows_a,
                                     sem_ga)

                pltpu.make_async_copy(table_hbm.at[src_v.at[b0 + 1]], rows_b,
                                      sem_gb).wait()
                pltpu.sync_copy(rows_b, acc.at[dst_v.at[b0 + 1]], add=True)
                return 0

            lax.fori_loop(0, HB // 2, pair, 0)
        plsc.subcore_barrier()
        pltpu.sync_copy(acc.at[pl.ds(s * RPS, RPS)],
                        out_hbm.at[c, pl.ds(s * RPS, RPS)])

    return prop


_prop128 = _make_prop(128)

RB = 1000
GRID = N // RB


def _tc_spec(d):
    return pl.BlockSpec((RB, d), lambda i: (i, 0))


def _full_spec(shape):
    return pl.BlockSpec(shape, lambda i: tuple(0 for _ in shape))


def _tc_stage1(deg0, deg1, x, m, pos, W1):
    def body(d0_r, d1_r, x_r, m_r, pos_r, w_r, dis_o, t1_o):
        deg = d0_r[..., 0] + d1_r[..., 0] + 1.0
        dis = (1.0 / jnp.sqrt(deg))[:, None]
        xb = x_r[...] + m_r[...] * pos_r[...]
        t1 = jnp.dot(xb, w_r[...], preferred_element_type=jnp.float32)
        dis_o[...] = dis
        t1_o[...] = t1 * dis

    return pl.pallas_call(
        body,
        grid=(GRID,),
        in_specs=[_tc_spec(128), _tc_spec(128), _tc_spec(128), _tc_spec(1),
                  _full_spec((1, 128)), _full_spec((128, 128))],
        out_specs=[_tc_spec(1), _tc_spec(128)],
        out_shape=[jax.ShapeDtypeStruct((N, 1), jnp.float32),
                   jax.ShapeDtypeStruct((N, 128), jnp.float32)],
    )(deg0, deg1, x, m, pos, W1)


def _tc_stage2(p0, p1, t1, dis, b1, a_enc, W2):
    def body(p0_r, p1_r, t1_r, dis_r, b_r, a_r, w_r, t2_o):
        a = a_r[0, 0]
        dis = dis_r[...]
        tot = dis * (p0_r[...] + p1_r[...] + t1_r[...]) + b_r[...]
        h = jnp.where(tot >= 0, tot, a * tot)
        t2 = dis * jnp.dot(h, w_r[...], preferred_element_type=jnp.float32)
        t2_o[...] = jnp.concatenate(
            [t2, jnp.zeros((t2.shape[0], 64), jnp.float32)], axis=1)

    return pl.pallas_call(
        body,
        grid=(GRID,),
        in_specs=[_tc_spec(128), _tc_spec(128), _tc_spec(128), _tc_spec(1),
                  _full_spec((1, 128)), _full_spec((1, 1)),
                  _full_spec((128, 64))],
        out_specs=_tc_spec(128),
        out_shape=jax.ShapeDtypeStruct((N, 128), jnp.float32),
    )(p0, p1, t1, dis, b1, a_enc, W2)


def _tc_stage3(p0, p1, t2, dis, b2, a_enc, We2d, Wd, m):
    def body(p0_r, p1_r, t2_r, dis_r, b_r, a_r, we_r, wd_r, m_r, t3_o):
        a = a_r[0, 0]
        dis = dis_r[...]
        tot = dis * (p0_r[...] + p1_r[...] + t2_r[...])[:, :64] + b_r[...]
        h2 = jnp.where(tot >= 0, tot, a * tot)
        rec = lax.dot_general(h2, we_r[...], (((1,), (1,)), ((), ())),
                              preferred_element_type=jnp.float32)
        rec = rec * (1.0 - m_r[...])
        t3_o[...] = dis * jnp.dot(rec, wd_r[...],
                                  preferred_element_type=jnp.float32)

    return pl.pallas_call(
        body,
        grid=(GRID,),
        in_specs=[_tc_spec(128), _tc_spec(128), _tc_spec(128), _tc_spec(1),
                  _full_spec((1, 64)), _full_spec((1, 1)),
                  _full_spec((64, 64)), _full_spec((64, 128)), _tc_spec(1)],
        out_specs=_tc_spec(128),
        out_shape=jax.ShapeDtypeStruct((N, 128), jnp.float32),
    )(p0, p1, t2, dis, b2, a_enc, We2d, Wd, m)


def _tc_stage4(p0, p1, t3, dis, bd, a_dec, x, m):
    def body(p0_r, p1_r, t3_r, dis_r, b_r, a_r, x_r, m_r, out_o):
        i = pl.program_id(0)
        a = a_r[0, 0]
        dis = dis_r[...]
        tot = dis * (p0_r[...] + p1_r[...] + t3_r[...]) + b_r[...]
        dec = jnp.where(tot >= 0, tot, a * tot)
        xv = x_r[...]
        xn = xv / jnp.maximum(
            jnp.sqrt(jnp.sum(xv * xv, axis=-1, keepdims=True)), 1e-12)
        rn = dec / jnp.maximum(
            jnp.sqrt(jnp.sum(dec * dec, axis=-1, keepdims=True)), 1e-12)
        cos = jnp.sum(xn * rn, axis=-1)
        e = 1.0 - cos
        contrib = jnp.sum(m_r[..., 0] * e * e) * (1.0 / NUM_MASK)

        @pl.when(i == 0)
        def _():
            out_o[...] = jnp.zeros((1, 1), jnp.float32)

        out_o[...] += jnp.reshape(contrib, (1, 1))

    return pl.pallas_call(
        body,
        grid=(GRID,),
        in_specs=[_tc_spec(128), _tc_spec(128), _tc_spec(128), _tc_spec(1),
                  _full_spec((1, 128)), _full_spec((1, 1)), _tc_spec(128),
                  _tc_spec(1)],
        out_specs=_full_spec((1, 1)),
        out_shape=jax.ShapeDtypeStruct((1, 1), jnp.float32),
    )(p0, p1, t3, dis, bd, a_dec, x, m)


def kernel(x, edge_index, W1, b1, W2, b2, a_enc, W_e2d, Wd, bd, a_dec,
           pos_token):
    f32 = jnp.float32
    ei = edge_index.astype(jnp.int32)
    npadlen = EPAD - E
    pad_i = jnp.arange(npadlen, dtype=jnp.int32)
    src_pad = jnp.concatenate([ei[0], (pad_i * 131) % N]).reshape(NW, NBLK, BLK)
    dst_pad = jnp.concatenate([ei[1], N + pad_i % (NPAD - N)]).reshape(
        NW, NBLK, BLK)

    perm = jax.random.permutation(jax.random.key(42), N)
    mask_nodes = perm[:NUM_MASK]
    m = jnp.zeros((N,), f32).at[mask_nodes].set(1.0).reshape(N, 1)

    degp = _deg_sc(dst_pad)
    dis, t1 = _tc_stage1(degp[0], degp[1], x, m, pos_token, W1)

    part1 = _prop128(t1, src_pad, dst_pad)
    t2 = _tc_stage2(part1[0], part1[1], t1, dis, b1.reshape(1, 128),
                    a_enc.reshape(1, 1), W2)

    part2 = _prop128(t2, src_pad, dst_pad)
    t3 = _tc_stage3(part2[0], part2[1], t2, dis, b2.reshape(1, 64),
                    a_enc.reshape(1, 1), W_e2d, Wd, m)

    part3 = _prop128(t3, src_pad, dst_pad)
    out = _tc_stage4(part3[0], part3[1], t3, dis, bd.reshape(1, 128),
                     a_dec.reshape(1, 1), x, m)
    return out[0, 0]


# speedup vs baseline: 1.2817x; 1.0559x over previous
import functools

import jax
import jax.numpy as jnp
from jax import lax
from jax.experimental import pallas as pl
from jax.experimental.pallas import tpu as pltpu
from jax.experimental.pallas import tpu_sc as plsc

N = 10000
E = 320000
NUM_MASK = 3000

NC = 2
NS = 16
NW = NC * NS
NPAD = 10240
EW = 10240
EPAD = NW * EW
BLK = 128
NBLK = EW // BLK
RPS = NPAD // NS

_mesh = plsc.VectorSubcoreMesh(core_axis_name="c", subcore_axis_name="s")


def _fill(ref, nrows, width, value):
    v = jnp.full((16,), value, jnp.float32)

    def row(i, _):
        for j in range(width // 16):
            ref[i, pl.ds(j * 16, 16)] = v
        return 0

    lax.fori_loop(0, nrows, row, 0)


@functools.partial(
    pl.kernel,
    out_type=jax.ShapeDtypeStruct((NC, NPAD, 128), jnp.float32),
    mesh=_mesh,
    scratch_types=[
        pltpu.VMEM((NBLK, BLK), jnp.int32),
        pltpu.VMEM((BLK, 128), jnp.float32),
        pltpu.VMEM((64, 128), jnp.float32),
        pltpu.VMEM_SHARED((NPAD, 128), jnp.float32),
        pltpu.SemaphoreType.DMA,
    ],
)
def _deg_sc(dst_hbm, out_hbm, dst_v, ones_v, zb_v, acc, sem):
    c = lax.axis_index("c")
    s = lax.axis_index("s")
    wid = s * NC + c
    _fill(ones_v, BLK, 128, 1.0)
    _fill(zb_v, 64, 128, 0.0)

    def zero(t, _):
        pltpu.sync_copy(zb_v, acc.at[pl.ds(s * RPS + t * 64, 64)])
        return 0

    lax.fori_loop(0, RPS // 64, zero, 0)
    pltpu.sync_copy(dst_hbm.at[wid], dst_v)
    plsc.subcore_barrier()

    def blk(b, _):
        pltpu.async_copy(ones_v, acc.at[dst_v.at[b]], sem, add=True)
        return 0

    lax.fori_loop(0, NBLK, blk, 0)

    def drain(b, _):
        pltpu.make_async_copy(ones_v, acc.at[dst_v.at[0]], sem).wait()
        return 0

    lax.fori_loop(0, NBLK, drain, 0)
    plsc.subcore_barrier()
    pltpu.sync_copy(acc.at[pl.ds(s * RPS, RPS)],
                    out_hbm.at[c, pl.ds(s * RPS, RPS)])


def _make_prop(D, tc_tiling=True):
    HB = NBLK // 2
    params = None if tc_tiling else pltpu.CompilerParams(
        use_tc_tiling_on_sc=False)

    @functools.partial(
        pl.kernel,
        out_type=jax.ShapeDtypeStruct((NC, NPAD, D), jnp.float32),
        mesh=_mesh,
        compiler_params=params,
        scratch_types=[
            pltpu.VMEM((HB, BLK), jnp.int32),
            pltpu.VMEM((HB, BLK), jnp.int32),
            pltpu.VMEM((BLK, D), jnp.float32),
            pltpu.VMEM((BLK, D), jnp.float32),
            pltpu.VMEM_SHARED((NPAD, D), jnp.float32),
            pltpu.SemaphoreType.DMA,
            pltpu.SemaphoreType.DMA,
        ],
    )
    def prop(table_hbm, src_hbm, dst_hbm, out_hbm, src_v, dst_v, rows_a,
             rows_b, acc, sem_ga, sem_gb):
        c = lax.axis_index("c")
        s = lax.axis_index("s")
        wid = s * NC + c
        _fill(rows_a, BLK, D, 0.0)

        def zero(t, _):
            pltpu.sync_copy(rows_a, acc.at[pl.ds(s * RPS + t * BLK, BLK)])
            return 0

        lax.fori_loop(0, RPS // BLK, zero, 0)
        plsc.subcore_barrier()

        for h in range(2):
            pltpu.sync_copy(src_hbm.at[wid, pl.ds(h * HB, HB)], src_v)
            pltpu.sync_copy(dst_hbm.at[wid, pl.ds(h * HB, HB)], dst_v)
            pltpu.async_copy(table_hbm.at[src_v.at[0]], rows_a, sem_ga)

            def pair(g, _):
                b0 = 2 * g
                pltpu.async_copy(table_hbm.at[src_v.at[b0 + 1]], rows_b,
                                 sem_gb)
                pltpu.make_async_copy(table_hbm.at[src_v.at[b0]], rows_a,
                                      sem_ga).wait()
                pltpu.sync_copy(rows_a, acc.at[dst_v.at[b0]], add=True)

                @pl.when(b0 + 2 < HB)
                def _():
                    pltpu.async_copy(table_hbm.at[src_v.at[b0 + 2]], rows_a,
                                     sem_ga)

                pltpu.make_async_copy(table_hbm.at[src_v.at[b0 + 1]], rows_b,
                                      sem_gb).wait()
                pltpu.sync_copy(rows_b, acc.at[dst_v.at[b0 + 1]], add=True)
                return 0

            lax.fori_loop(0, HB // 2, pair, 0)
        plsc.subcore_barrier()
        pltpu.sync_copy(acc.at[pl.ds(s * RPS, RPS)],
                        out_hbm.at[c, pl.ds(s * RPS, RPS)])

    return prop


_prop128 = _make_prop(128)
_prop64 = _make_prop(64, tc_tiling=False)

RB = 1000
GRID = N // RB


def _tc_spec(d):
    return pl.BlockSpec((RB, d), lambda i: (i, 0))


def _full_spec(shape):
    return pl.BlockSpec(shape, lambda i: tuple(0 for _ in shape))


def _tc_stage1(deg0, deg1, x, m, pos, W1):
    def body(d0_r, d1_r, x_r, m_r, pos_r, w_r, dis_o, t1_o):
        deg = d0_r[..., 0] + d1_r[..., 0] + 1.0
        dis = (1.0 / jnp.sqrt(deg))[:, None]
        xb = x_r[...] + m_r[...] * pos_r[...]
        t1 = jnp.dot(xb, w_r[...], preferred_element_type=jnp.float32)
        dis_o[...] = dis
        t1_o[...] = t1 * dis

    return pl.pallas_call(
        body,
        grid=(GRID,),
        in_specs=[_tc_spec(128), _tc_spec(128), _tc_spec(128), _tc_spec(1),
                  _full_spec((1, 128)), _full_spec((128, 128))],
        out_specs=[_tc_spec(1), _tc_spec(128)],
        out_shape=[jax.ShapeDtypeStruct((N, 1), jnp.float32),
                   jax.ShapeDtypeStruct((N, 128), jnp.float32)],
    )(deg0, deg1, x, m, pos, W1)


def _tc_stage2(p0, p1, t1, dis, b1, a_enc, W2):
    def body(p0_r, p1_r, t1_r, dis_r, b_r, a_r, w_r, t2_o):
        a = a_r[0, 0]
        dis = dis_r[...]
        tot = dis * (p0_r[...] + p1_r[...] + t1_r[...]) + b_r[...]
        h = jnp.where(tot >= 0, tot, a * tot)
        t2_o[...] = dis * jnp.dot(h, w_r[...],
                                  preferred_element_type=jnp.float32)

    return pl.pallas_call(
        body,
        grid=(GRID,),
        in_specs=[_tc_spec(128), _tc_spec(128), _tc_spec(128), _tc_spec(1),
                  _full_spec((1, 128)), _full_spec((1, 1)),
                  _full_spec((128, 64))],
        out_specs=_tc_spec(64),
        out_shape=jax.ShapeDtypeStruct((N, 64), jnp.float32),
    )(p0, p1, t1, dis, b1, a_enc, W2)


def _tc_stage3(p0, p1, t2, dis, b2, a_enc, We2d, Wd, m):
    def body(p0_r, p1_r, t2_r, dis_r, b_r, a_r, we_r, wd_r, m_r, t3_o):
        a = a_r[0, 0]
        dis = dis_r[...]
        tot = dis * (p0_r[...] + p1_r[...] + t2_r[...]) + b_r[...]
        h2 = jnp.where(tot >= 0, tot, a * tot)
        rec = lax.dot_general(h2, we_r[...], (((1,), (1,)), ((), ())),
                              preferred_element_type=jnp.float32)
        rec = rec * (1.0 - m_r[...])
        t3_o[...] = dis * jnp.dot(rec, wd_r[...],
                                  preferred_element_type=jnp.float32)

    return pl.pallas_call(
        body,
        grid=(GRID,),
        in_specs=[_tc_spec(64), _tc_spec(64), _tc_spec(64), _tc_spec(1),
                  _full_spec((1, 64)), _full_spec((1, 1)),
                  _full_spec((64, 64)), _full_spec((64, 128)), _tc_spec(1)],
        out_specs=_tc_spec(128),
        out_shape=jax.ShapeDtypeStruct((N, 128), jnp.float32),
    )(p0, p1, t2, dis, b2, a_enc, We2d, Wd, m)


def _tc_stage4(p0, p1, t3, dis, bd, a_dec, x, m):
    def body(p0_r, p1_r, t3_r, dis_r, b_r, a_r, x_r, m_r, out_o):
        i = pl.program_id(0)
        a = a_r[0, 0]
        dis = dis_r[...]
        tot = dis * (p0_r[...] + p1_r[...] + t3_r[...]) + b_r[...]
        dec = jnp.where(tot >= 0, tot, a * tot)
        xv = x_r[...]
        xn = xv / jnp.maximum(
            jnp.sqrt(jnp.sum(xv * xv, axis=-1, keepdims=True)), 1e-12)
        rn = dec / jnp.maximum(
            jnp.sqrt(jnp.sum(dec * dec, axis=-1, keepdims=True)), 1e-12)
        cos = jnp.sum(xn * rn, axis=-1)
        e = 1.0 - cos
        contrib = jnp.sum(m_r[..., 0] * e * e) * (1.0 / NUM_MASK)

        @pl.when(i == 0)
        def _():
            out_o[...] = jnp.zeros((1, 1), jnp.float32)

        out_o[...] += jnp.reshape(contrib, (1, 1))

    return pl.pallas_call(
        body,
        grid=(GRID,),
        in_specs=[_tc_spec(128), _tc_spec(128), _tc_spec(128), _tc_spec(1),
                  _full_spec((1, 128)), _full_spec((1, 1)), _tc_spec(128),
                  _tc_spec(1)],
        out_specs=_full_spec((1, 1)),
        out_shape=jax.ShapeDtypeStruct((1, 1), jnp.float32),
    )(p0, p1, t3, dis, bd, a_dec, x, m)


def kernel(x, edge_index, W1, b1, W2, b2, a_enc, W_e2d, Wd, bd, a_dec,
           pos_token):
    f32 = jnp.float32
    ei = edge_index.astype(jnp.int32)
    npadlen = EPAD - E
    pad_i = jnp.arange(npadlen, dtype=jnp.int32)
    src_pad = jnp.concatenate([ei[0], (pad_i * 131) % N]).reshape(NW, NBLK, BLK)
    dst_pad = jnp.concatenate([ei[1], N + pad_i % (NPAD - N)]).reshape(
        NW, NBLK, BLK)

    perm = jax.random.permutation(jax.random.key(42), N)
    mask_nodes = perm[:NUM_MASK]
    m = jnp.zeros((N,), f32).at[mask_nodes].set(1.0).reshape(N, 1)

    degp = _deg_sc(dst_pad)
    dis, t1 = _tc_stage1(degp[0], degp[1], x, m, pos_token, W1)

    part1 = _prop128(t1, src_pad, dst_pad)
    t2 = _tc_stage2(part1[0], part1[1], t1, dis, b1.reshape(1, 128),
                    a_enc.reshape(1, 1), W2)

    part2 = _prop64(t2, src_pad, dst_pad)
    t3 = _tc_stage3(part2[0], part2[1], t2, dis, b2.reshape(1, 64),
                    a_enc.reshape(1, 1), W_e2d, Wd, m)

    part3 = _prop128(t3, src_pad, dst_pad)
    out = _tc_stage4(part3[0], part3[1], t3, dis, bd.reshape(1, 128),
                     a_dec.reshape(1, 1), x, m)
    return out[0, 0]


# speedup vs baseline: 1.3512x; 1.0542x over previous
import functools

import jax
import jax.numpy as jnp
from jax import lax
from jax.experimental import pallas as pl
from jax.experimental.pallas import tpu as pltpu
from jax.experimental.pallas import tpu_sc as plsc

N = 10000
E = 320000
NUM_MASK = 3000

NC = 2
NS = 16
NW = NC * NS
NPAD = 10240
EW = 10240
EPAD = NW * EW
BLK = 128
NBLK = EW // BLK
RPS = NPAD // NS

_mesh = plsc.VectorSubcoreMesh(core_axis_name="c", subcore_axis_name="s")


def _fill(ref, nrows, width, value):
    v = jnp.full((16,), value, jnp.float32)

    def row(i, _):
        for j in range(width // 16):
            ref[i, pl.ds(j * 16, 16)] = v
        return 0

    lax.fori_loop(0, nrows, row, 0)


@functools.partial(
    pl.kernel,
    out_type=jax.ShapeDtypeStruct((NC, NPAD), jnp.float32),
    mesh=_mesh,
    compiler_params=pltpu.CompilerParams(needs_layout_passes=False),
    scratch_types=[
        pltpu.VMEM((EW,), jnp.int32),
        pltpu.VMEM((NPAD,), jnp.float32),
        pltpu.VMEM((NS, RPS), jnp.float32),
        pltpu.VMEM_SHARED((NS, NPAD), jnp.float32),
    ],
)
def _deg_sc(dst_hbm, out_hbm, dst_v, hist, red_v, shared):
    c = lax.axis_index("c")
    s = lax.axis_index("s")
    wid = s * NC + c
    zero16 = jnp.zeros((16,), jnp.float32)
    ones16 = jnp.ones((16,), jnp.float32)

    def zr(i, _):
        hist[pl.ds(i * 16, 16)] = zero16
        return 0

    lax.fori_loop(0, NPAD // 16, zr, 0)
    pltpu.sync_copy(dst_hbm.at[wid], dst_v)

    def acc16(i, _):
        idx = dst_v[pl.ds(i * 16, 16)]
        plsc.addupdate_scatter(hist, [idx], ones16)
        return 0

    lax.fori_loop(0, EW // 16, acc16, 0)
    pltpu.sync_copy(hist, shared.at[s])
    plsc.subcore_barrier()
    pltpu.sync_copy(shared.at[:, pl.ds(s * RPS, RPS)], red_v)

    def red(j, _):
        acc = zero16
        for r in range(NS):
            acc = acc + red_v[r, pl.ds(j * 16, 16)]
        hist[pl.ds(j * 16, 16)] = acc
        return 0

    lax.fori_loop(0, RPS // 16, red, 0)
    pltpu.sync_copy(hist.at[pl.ds(0, RPS)], out_hbm.at[c, pl.ds(s * RPS, RPS)])


def _make_prop(D, tc_tiling=True, bpg=1):
    NG = NBLK // bpg
    HG = NG // 2
    GE = bpg * BLK
    params = None if tc_tiling else pltpu.CompilerParams(
        use_tc_tiling_on_sc=False)

    @functools.partial(
        pl.kernel,
        out_type=jax.ShapeDtypeStruct((NC, NPAD, D), jnp.float32),
        mesh=_mesh,
        compiler_params=params,
        scratch_types=[
            pltpu.VMEM((HG, GE), jnp.int32),
            pltpu.VMEM((HG, GE), jnp.int32),
            pltpu.VMEM((GE, D), jnp.float32),
            pltpu.VMEM((GE, D), jnp.float32),
            pltpu.VMEM_SHARED((NPAD, D), jnp.float32),
            pltpu.SemaphoreType.DMA,
            pltpu.SemaphoreType.DMA,
        ],
    )
    def prop(table_hbm, src_hbm, dst_hbm, out_hbm, src_v, dst_v, rows_a,
             rows_b, acc, sem_ga, sem_gb):
        c = lax.axis_index("c")
        s = lax.axis_index("s")
        wid = s * NC + c
        _fill(rows_a, GE, D, 0.0)

        def zero(t, _):
            pltpu.sync_copy(rows_a.at[pl.ds(0, BLK)],
                            acc.at[pl.ds(s * RPS + t * BLK, BLK)])
            return 0

        lax.fori_loop(0, RPS // BLK, zero, 0)
        plsc.subcore_barrier()

        for h in range(2):
            pltpu.sync_copy(src_hbm.at[wid, pl.ds(h * HG, HG)], src_v)
            pltpu.sync_copy(dst_hbm.at[wid, pl.ds(h * HG, HG)], dst_v)
            pltpu.async_copy(table_hbm.at[src_v.at[0]], rows_a, sem_ga)

            def pair(g, _):
                b0 = 2 * g
                pltpu.async_copy(table_hbm.at[src_v.at[b0 + 1]], rows_b,
                                 sem_gb)
                pltpu.make_async_copy(table_hbm.at[src_v.at[b0]], rows_a,
                                      sem_ga).wait()
                pltpu.sync_copy(rows_a, acc.at[dst_v.at[b0]], add=True)

                @pl.when(b0 + 2 < HG)
                def _():
                    pltpu.async_copy(table_hbm.at[src_v.at[b0 + 2]], rows_a,
                                     sem_ga)

                pltpu.make_async_copy(table_hbm.at[src_v.at[b0 + 1]], rows_b,
                                      sem_gb).wait()
                pltpu.sync_copy(rows_b, acc.at[dst_v.at[b0 + 1]], add=True)
                return 0

            lax.fori_loop(0, HG // 2, pair, 0)
        plsc.subcore_barrier()
        pltpu.sync_copy(acc.at[pl.ds(s * RPS, RPS)],
                        out_hbm.at[c, pl.ds(s * RPS, RPS)])

    return prop


_prop128 = _make_prop(128)
_prop64 = _make_prop(64, tc_tiling=False, bpg=2)

RB = 1000
GRID = N // RB


def _tc_spec(d):
    return pl.BlockSpec((RB, d), lambda i: (i, 0))


def _full_spec(shape):
    return pl.BlockSpec(shape, lambda i: tuple(0 for _ in shape))


def _tc_dis(degp):
    DB = 1280

    def body(d_r, dis_o):
        deg = d_r[0, :] + d_r[1, :] + 1.0
        dis_o[...] = (1.0 / jnp.sqrt(deg))[:, None]

    return pl.pallas_call(
        body,
        grid=(NPAD // DB,),
        in_specs=[pl.BlockSpec((NC, DB), lambda i: (0, i))],
        out_specs=pl.BlockSpec((DB, 1), lambda i: (i, 0)),
        out_shape=jax.ShapeDtypeStruct((NPAD, 1), jnp.float32),
    )(degp)


def _tc_stage1(dis, x, m, pos, W1):
    def body(dis_r, x_r, m_r, pos_r, w_r, t1_o):
        xb = x_r[...] + m_r[...] * pos_r[...]
        t1 = jnp.dot(xb, w_r[...], preferred_element_type=jnp.float32)
        t1_o[...] = t1 * dis_r[...]

    return pl.pallas_call(
        body,
        grid=(GRID,),
        in_specs=[_tc_spec(1), _tc_spec(128), _tc_spec(1),
                  _full_spec((1, 128)), _full_spec((128, 128))],
        out_specs=_tc_spec(128),
        out_shape=jax.ShapeDtypeStruct((N, 128), jnp.float32),
    )(dis, x, m, pos, W1)


def _tc_stage2(p0, p1, t1, dis, b1, a_enc, W2):
    def body(p0_r, p1_r, t1_r, dis_r, b_r, a_r, w_r, t2_o):
        a = a_r[0, 0]
        dis = dis_r[...]
        tot = dis * (p0_r[...] + p1_r[...] + t1_r[...]) + b_r[...]
        h = jnp.where(tot >= 0, tot, a * tot)
        t2_o[...] = dis * jnp.dot(h, w_r[...],
                                  preferred_element_type=jnp.float32)

    return pl.pallas_call(
        body,
        grid=(GRID,),
        in_specs=[_tc_spec(128), _tc_spec(128), _tc_spec(128), _tc_spec(1),
                  _full_spec((1, 128)), _full_spec((1, 1)),
                  _full_spec((128, 64))],
        out_specs=_tc_spec(64),
        out_shape=jax.ShapeDtypeStruct((N, 64), jnp.float32),
    )(p0, p1, t1, dis, b1, a_enc, W2)


def _tc_stage3(p0, p1, t2, dis, b2, a_enc, We2d, Wd, m):
    def body(p0_r, p1_r, t2_r, dis_r, b_r, a_r, we_r, wd_r, m_r, t3_o):
        a = a_r[0, 0]
        dis = dis_r[...]
        tot = dis * (p0_r[...] + p1_r[...] + t2_r[...]) + b_r[...]
        h2 = jnp.where(tot >= 0, tot, a * tot)
        rec = lax.dot_general(h2, we_r[...], (((1,), (1,)), ((), ())),
                              preferred_element_type=jnp.float32)
        rec = rec * (1.0 - m_r[...])
        t3_o[...] = dis * jnp.dot(rec, wd_r[...],
                                  preferred_element_type=jnp.float32)

    return pl.pallas_call(
        body,
        grid=(GRID,),
        in_specs=[_tc_spec(64), _tc_spec(64), _tc_spec(64), _tc_spec(1),
                  _full_spec((1, 64)), _full_spec((1, 1)),
                  _full_spec((64, 64)), _full_spec((64, 128)), _tc_spec(1)],
        out_specs=_tc_spec(128),
        out_shape=jax.ShapeDtypeStruct((N, 128), jnp.float32),
    )(p0, p1, t2, dis, b2, a_enc, We2d, Wd, m)


def _tc_stage4(p0, p1, t3, dis, bd, a_dec, x, m):
    def body(p0_r, p1_r, t3_r, dis_r, b_r, a_r, x_r, m_r, out_o):
        i = pl.program_id(0)
        a = a_r[0, 0]
        dis = dis_r[...]
        tot = dis * (p0_r[...] + p1_r[...] + t3_r[...]) + b_r[...]
        dec = jnp.where(tot >= 0, tot, a * tot)
        xv = x_r[...]
        xn = xv / jnp.maximum(
            jnp.sqrt(jnp.sum(xv * xv, axis=-1, keepdims=True)), 1e-12)
        rn = dec / jnp.maximum(
            jnp.sqrt(jnp.sum(dec * dec, axis=-1, keepdims=True)), 1e-12)
        cos = jnp.sum(xn * rn, axis=-1)
        e = 1.0 - cos
        contrib = jnp.sum(m_r[..., 0] * e * e) * (1.0 / NUM_MASK)

        @pl.when(i == 0)
        def _():
            out_o[...] = jnp.zeros((1, 1), jnp.float32)

        out_o[...] += jnp.reshape(contrib, (1, 1))

    return pl.pallas_call(
        body,
        grid=(GRID,),
        in_specs=[_tc_spec(128), _tc_spec(128), _tc_spec(128), _tc_spec(1),
                  _full_spec((1, 128)), _full_spec((1, 1)), _tc_spec(128),
                  _tc_spec(1)],
        out_specs=_full_spec((1, 1)),
        out_shape=jax.ShapeDtypeStruct((1, 1), jnp.float32),
    )(p0, p1, t3, dis, bd, a_dec, x, m)


def kernel(x, edge_index, W1, b1, W2, b2, a_enc, W_e2d, Wd, bd, a_dec,
           pos_token):
    f32 = jnp.float32
    ei = edge_index.astype(jnp.int32)
    npadlen = EPAD - E
    pad_i = jnp.arange(npadlen, dtype=jnp.int32)
    src_pad = jnp.concatenate([ei[0], (pad_i * 131) % N]).reshape(NW, NBLK, BLK)
    dst_pad = jnp.concatenate([ei[1], N + pad_i % (NPAD - N)]).reshape(
        NW, NBLK, BLK)
    src_pad2 = src_pad.reshape(NW, NBLK // 2, 2 * BLK)
    dst_pad2 = dst_pad.reshape(NW, NBLK // 2, 2 * BLK)

    perm = jax.random.permutation(jax.random.key(42), N)
    mask_nodes = perm[:NUM_MASK]
    m = jnp.zeros((N,), f32).at[mask_nodes].set(1.0).reshape(N, 1)

    degp = _deg_sc(dst_pad.reshape(NW, EW))
    dis = _tc_dis(degp)
    t1 = _tc_stage1(dis, x, m, pos_token, W1)

    src_pad1 = src_pad
    dst_pad1 = dst_pad
    part1 = _prop128(t1, src_pad1, dst_pad1)
    t2 = _tc_stage2(part1[0], part1[1], t1, dis, b1.reshape(1, 128),
                    a_enc.reshape(1, 1), W2)

    part2 = _prop64(t2, src_pad2, dst_pad2)
    t3 = _tc_stage3(part2[0], part2[1], t2, dis, b2.reshape(1, 64),
                    a_enc.reshape(1, 1), W_e2d, Wd, m)

    part3 = _prop128(t3, src_pad1, dst_pad1)
    out = _tc_stage4(part3[0], part3[1], t3, dis, bd.reshape(1, 128),
                     a_dec.reshape(1, 1), x, m)
    return out[0, 0]


# speedup vs baseline: 1.4582x; 1.0792x over previous
import functools

import jax
import jax.numpy as jnp
from jax import lax
from jax.experimental import pallas as pl
from jax.experimental.pallas import tpu as pltpu
from jax.experimental.pallas import tpu_sc as plsc

N = 10000
E = 320000
NUM_MASK = 3000

NC = 2
NS = 16
NW = NC * NS
NPAD = 10240
EW = 10240
EPAD = NW * EW
BLK = 128
NBLK = EW // BLK
RPS = NPAD // NS

_mesh = plsc.VectorSubcoreMesh(core_axis_name="c", subcore_axis_name="s")


def _fill(ref, nrows, width, value):
    v = jnp.full((16,), value, jnp.float32)

    def row(i, _):
        for j in range(width // 16):
            ref[i, pl.ds(j * 16, 16)] = v
        return 0

    lax.fori_loop(0, nrows, row, 0)


@functools.partial(
    pl.kernel,
    out_type=[jax.ShapeDtypeStruct((NC, NPAD), jnp.float32),
              jax.ShapeDtypeStruct((NW, EW), jnp.int32),
              jax.ShapeDtypeStruct((NW, EW), jnp.int32),
              jax.ShapeDtypeStruct((NW, 16), jnp.int32)],
    mesh=_mesh,
    compiler_params=pltpu.CompilerParams(needs_layout_passes=False),
    scratch_types=[
        pltpu.VMEM((EW,), jnp.int32),
        pltpu.VMEM((EW,), jnp.int32),
        pltpu.VMEM((NPAD,), jnp.float32),
        pltpu.VMEM((EW,), jnp.int32),
        pltpu.VMEM((EW,), jnp.int32),
        pltpu.VMEM((16,), jnp.int32),
        pltpu.VMEM((NPAD,), jnp.float32),
        pltpu.VMEM((NS, RPS), jnp.float32),
        pltpu.VMEM_SHARED((NS, NPAD), jnp.float32),
    ],
)
def _deg_sc(dst_hbm, src_hbm, mask_hbm, deg_out, fsrc_out, fdst_out,
            fcnt_out, dst_v, src_v, mask_v, fdst_v, fsrc_v, cnt_v, hist,
            red_v, shared):
    c = lax.axis_index("c")
    s = lax.axis_index("s")
    wid = s * NC + c
    zero16 = jnp.zeros((16,), jnp.float32)
    ones16 = jnp.ones((16,), jnp.float32)
    iota16 = lax.iota(jnp.int32, 16)

    def zr(i, _):
        hist[pl.ds(i * 16, 16)] = zero16
        return 0

    lax.fori_loop(0, NPAD // 16, zr, 0)
    pltpu.sync_copy(dst_hbm.at[wid], dst_v)
    pltpu.sync_copy(src_hbm.at[wid], src_v)
    pltpu.sync_copy(mask_hbm, mask_v)

    def acc16(i, off_vec):
        idx = dst_v[pl.ds(i * 16, 16)]
        plsc.addupdate_scatter(hist, [idx], ones16)
        sv = src_v[pl.ds(i * 16, 16)]
        fl = plsc.load_gather(mask_v, [idx]) > 0.0
        cum = plsc.cumsum(fl.astype(jnp.int32))
        pos = off_vec + cum - 1
        plsc.store_scatter(fdst_v, [pos], idx, mask=fl)
        plsc.store_scatter(fsrc_v, [pos], sv, mask=fl)
        return off_vec + plsc.all_reduce_population_count(fl)

    off_vec = lax.fori_loop(0, EW // 16, acc16, jnp.zeros((16,), jnp.int32))
    target = ((off_vec + 255) // 256) * 256
    for k in range(16):
        pos = off_vec + k * 16 + iota16
        mk = pos < target
        plsc.store_scatter(fdst_v, [pos], N + iota16, mask=mk)
        plsc.store_scatter(fsrc_v, [pos], iota16, mask=mk)
    cnt_v[pl.ds(0, 16)] = target // 256
    pltpu.sync_copy(fsrc_v, fsrc_out.at[wid])
    pltpu.sync_copy(fdst_v, fdst_out.at[wid])
    pltpu.sync_copy(cnt_v, fcnt_out.at[wid])

    pltpu.sync_copy(hist, shared.at[s])
    plsc.subcore_barrier()
    pltpu.sync_copy(shared.at[:, pl.ds(s * RPS, RPS)], red_v)

    def red(j, _):
        acc = zero16
        for r in range(NS):
            acc = acc + red_v[r, pl.ds(j * 16, 16)]
        hist[pl.ds(j * 16, 16)] = acc
        return 0

    lax.fori_loop(0, RPS // 16, red, 0)
    pltpu.sync_copy(hist.at[pl.ds(0, RPS)],
                    deg_out.at[c, pl.ds(s * RPS, RPS)])


def _make_prop(D, tc_tiling=True, bpg=1):
    NG = NBLK // bpg
    HG = NG // 2
    GE = bpg * BLK
    params = None if tc_tiling else pltpu.CompilerParams(
        use_tc_tiling_on_sc=False)

    @functools.partial(
        pl.kernel,
        out_type=jax.ShapeDtypeStruct((NC, NPAD, D), jnp.float32),
        mesh=_mesh,
        compiler_params=params,
        scratch_types=[
            pltpu.VMEM((HG, GE), jnp.int32),
            pltpu.VMEM((HG, GE), jnp.int32),
            pltpu.VMEM((GE, D), jnp.float32),
            pltpu.VMEM((GE, D), jnp.float32),
            pltpu.VMEM_SHARED((NPAD, D), jnp.float32),
            pltpu.SemaphoreType.DMA,
            pltpu.SemaphoreType.DMA,
        ],
    )
    def prop(table_hbm, src_hbm, dst_hbm, out_hbm, src_v, dst_v, rows_a,
             rows_b, acc, sem_ga, sem_gb):
        c = lax.axis_index("c")
        s = lax.axis_index("s")
        wid = s * NC + c
        _fill(rows_a, GE, D, 0.0)

        def zero(t, _):
            pltpu.sync_copy(rows_a.at[pl.ds(0, BLK)],
                            acc.at[pl.ds(s * RPS + t * BLK, BLK)])
            return 0

        lax.fori_loop(0, RPS // BLK, zero, 0)
        plsc.subcore_barrier()

        for h in range(2):
            pltpu.sync_copy(src_hbm.at[wid, pl.ds(h * HG, HG)], src_v)
            pltpu.sync_copy(dst_hbm.at[wid, pl.ds(h * HG, HG)], dst_v)
            pltpu.async_copy(table_hbm.at[src_v.at[0]], rows_a, sem_ga)

            def pair(g, _):
                b0 = 2 * g
                pltpu.async_copy(table_hbm.at[src_v.at[b0 + 1]], rows_b,
                                 sem_gb)
                pltpu.make_async_copy(table_hbm.at[src_v.at[b0]], rows_a,
                                      sem_ga).wait()
                pltpu.sync_copy(rows_a, acc.at[dst_v.at[b0]], add=True)

                @pl.when(b0 + 2 < HG)
                def _():
                    pltpu.async_copy(table_hbm.at[src_v.at[b0 + 2]], rows_a,
                                     sem_ga)

                pltpu.make_async_copy(table_hbm.at[src_v.at[b0 + 1]], rows_b,
                                      sem_gb).wait()
                pltpu.sync_copy(rows_b, acc.at[dst_v.at[b0 + 1]], add=True)
                return 0

            lax.fori_loop(0, HG // 2, pair, 0)
        plsc.subcore_barrier()
        pltpu.sync_copy(acc.at[pl.ds(s * RPS, RPS)],
                        out_hbm.at[c, pl.ds(s * RPS, RPS)])

    return prop


def _make_prop_masked(D):
    HG = NBLK // 2

    @functools.partial(
        pl.kernel,
        out_type=jax.ShapeDtypeStruct((NC, NPAD, D), jnp.float32),
        mesh=_mesh,
        scratch_types=[
            pltpu.VMEM((HG, BLK), jnp.int32),
            pltpu.VMEM((HG, BLK), jnp.int32),
            pltpu.VMEM((BLK, D), jnp.float32),
            pltpu.VMEM((BLK, D), jnp.float32),
            pltpu.VMEM((16,), jnp.int32),
            pltpu.VMEM_SHARED((NPAD, D), jnp.float32),
            pltpu.SemaphoreType.DMA,
            pltpu.SemaphoreType.DMA,
        ],
    )
    def prop(table_hbm, src_hbm, dst_hbm, cnt_hbm, out_hbm, src_v, dst_v,
             rows_a, rows_b, cnt_v, acc, sem_ga, sem_gb):
        c = lax.axis_index("c")
        s = lax.axis_index("s")
        wid = s * NC + c
        pltpu.sync_copy(cnt_hbm.at[wid], cnt_v)
        _fill(rows_a, BLK, D, 0.0)

        def zero(t, _):
            pltpu.sync_copy(rows_a.at[pl.ds(0, BLK)],
                            acc.at[pl.ds(s * RPS + t * BLK, BLK)])
            return 0

        lax.fori_loop(0, RPS // BLK, zero, 0)
        plsc.subcore_barrier()

        npairs = cnt_v[pl.ds(0, 16)][0]
        p0 = jnp.minimum(npairs, HG // 2)
        for h, ph in ((0, p0), (1, npairs - p0)):
            pltpu.sync_copy(src_hbm.at[wid, pl.ds(h * HG, HG)], src_v)
            pltpu.sync_copy(dst_hbm.at[wid, pl.ds(h * HG, HG)], dst_v)

            @pl.when(ph > 0)
            def _():
                pltpu.async_copy(table_hbm.at[src_v.at[0]], rows_a, sem_ga)

            def pair(g, _):
                b0 = 2 * g
                pltpu.async_copy(table_hbm.at[src_v.at[b0 + 1]], rows_b,
                                 sem_gb)
                pltpu.make_async_copy(table_hbm.at[src_v.at[b0]], rows_a,
                                      sem_ga).wait()
                pltpu.sync_copy(rows_a, acc.at[dst_v.at[b0]], add=True)

                @pl.when(b0 + 2 < 2 * ph)
                def _():
                    pltpu.async_copy(table_hbm.at[src_v.at[b0 + 2]], rows_a,
                                     sem_ga)

                pltpu.make_async_copy(table_hbm.at[src_v.at[b0 + 1]], rows_b,
                                      sem_gb).wait()
                pltpu.sync_copy(rows_b, acc.at[dst_v.at[b0 + 1]], add=True)
                return 0

            lax.fori_loop(0, ph, pair, 0)
        plsc.subcore_barrier()
        pltpu.sync_copy(acc.at[pl.ds(s * RPS, RPS)],
                        out_hbm.at[c, pl.ds(s * RPS, RPS)])

    return prop


_prop128 = _make_prop(128)
_prop_masked128 = _make_prop_masked(128)
_prop64 = _make_prop(64, tc_tiling=False, bpg=2)

RB = 1000
GRID = N // RB


def _tc_spec(d):
    return pl.BlockSpec((RB, d), lambda i: (i, 0))


def _full_spec(shape):
    return pl.BlockSpec(shape, lambda i: tuple(0 for _ in shape))


def _tc_dis(degp):
    DB = 1280

    def body(d_r, dis_o):
        deg = d_r[0, :] + d_r[1, :] + 1.0
        dis_o[...] = (1.0 / jnp.sqrt(deg))[:, None]

    return pl.pallas_call(
        body,
        grid=(NPAD // DB,),
        in_specs=[pl.BlockSpec((NC, DB), lambda i: (0, i))],
        out_specs=pl.BlockSpec((DB, 1), lambda i: (i, 0)),
        out_shape=jax.ShapeDtypeStruct((NPAD, 1), jnp.float32),
    )(degp)


def _tc_stage1(dis, x, m, pos, W1):
    def body(dis_r, x_r, m_r, pos_r, w_r, t1_o):
        xb = x_r[...] + m_r[...] * pos_r[...]
        t1 = jnp.dot(xb, w_r[...], preferred_element_type=jnp.float32)
        t1_o[...] = t1 * dis_r[...]

    return pl.pallas_call(
        body,
        grid=(GRID,),
        in_specs=[_tc_spec(1), _tc_spec(128), _tc_spec(1),
                  _full_spec((1, 128)), _full_spec((128, 128))],
        out_specs=_tc_spec(128),
        out_shape=jax.ShapeDtypeStruct((N, 128), jnp.float32),
    )(dis, x, m, pos, W1)


def _tc_stage2(p0, p1, t1, dis, b1, a_enc, W2):
    def body(p0_r, p1_r, t1_r, dis_r, b_r, a_r, w_r, t2_o):
        a = a_r[0, 0]
        dis = dis_r[...]
        tot = dis * (p0_r[...] + p1_r[...] + t1_r[...]) + b_r[...]
        h = jnp.where(tot >= 0, tot, a * tot)
        t2_o[...] = dis * jnp.dot(h, w_r[...],
                                  preferred_element_type=jnp.float32)

    return pl.pallas_call(
        body,
        grid=(GRID,),
        in_specs=[_tc_spec(128), _tc_spec(128), _tc_spec(128), _tc_spec(1),
                  _full_spec((1, 128)), _full_spec((1, 1)),
                  _full_spec((128, 64))],
        out_specs=_tc_spec(64),
        out_shape=jax.ShapeDtypeStruct((N, 64), jnp.float32),
    )(p0, p1, t1, dis, b1, a_enc, W2)


def _tc_stage3(p0, p1, t2, dis, b2, a_enc, We2d, Wd, m):
    def body(p0_r, p1_r, t2_r, dis_r, b_r, a_r, we_r, wd_r, m_r, t3_o):
        a = a_r[0, 0]
        dis = dis_r[...]
        tot = dis * (p0_r[...] + p1_r[...] + t2_r[...]) + b_r[...]
        h2 = jnp.where(tot >= 0, tot, a * tot)
        rec = lax.dot_general(h2, we_r[...], (((1,), (1,)), ((), ())),
                              preferred_element_type=jnp.float32)
        rec = rec * (1.0 - m_r[...])
        t3_o[...] = dis * jnp.dot(rec, wd_r[...],
                                  preferred_element_type=jnp.float32)

    return pl.pallas_call(
        body,
        grid=(GRID,),
        in_specs=[_tc_spec(64), _tc_spec(64), _tc_spec(64), _tc_spec(1),
                  _full_spec((1, 64)), _full_spec((1, 1)),
                  _full_spec((64, 64)), _full_spec((64, 128)), _tc_spec(1)],
        out_specs=_tc_spec(128),
        out_shape=jax.ShapeDtypeStruct((N, 128), jnp.float32),
    )(p0, p1, t2, dis, b2, a_enc, We2d, Wd, m)


def _tc_stage4(p0, p1, t3, dis, bd, a_dec, x, m):
    def body(p0_r, p1_r, t3_r, dis_r, b_r, a_r, x_r, m_r, out_o):
        i = pl.program_id(0)
        a = a_r[0, 0]
        dis = dis_r[...]
        tot = dis * (p0_r[...] + p1_r[...] + t3_r[...]) + b_r[...]
        dec = jnp.where(tot >= 0, tot, a * tot)
        xv = x_r[...]
        xn = xv / jnp.maximum(
            jnp.sqrt(jnp.sum(xv * xv, axis=-1, keepdims=True)), 1e-12)
        rn = dec / jnp.maximum(
            jnp.sqrt(jnp.sum(dec * dec, axis=-1, keepdims=True)), 1e-12)
        cos = jnp.sum(xn * rn, axis=-1)
        e = 1.0 - cos
        contrib = jnp.sum(m_r[..., 0] * e * e) * (1.0 / NUM_MASK)

        @pl.when(i == 0)
        def _():
            out_o[...] = jnp.zeros((1, 1), jnp.float32)

        out_o[...] += jnp.reshape(contrib, (1, 1))

    return pl.pallas_call(
        body,
        grid=(GRID,),
        in_specs=[_tc_spec(128), _tc_spec(128), _tc_spec(128), _tc_spec(1),
                  _full_spec((1, 128)), _full_spec((1, 1)), _tc_spec(128),
                  _tc_spec(1)],
        out_specs=_full_spec((1, 1)),
        out_shape=jax.ShapeDtypeStruct((1, 1), jnp.float32),
    )(p0, p1, t3, dis, bd, a_dec, x, m)


def kernel(x, edge_index, W1, b1, W2, b2, a_enc, W_e2d, Wd, bd, a_dec,
           pos_token):
    f32 = jnp.float32
    ei = edge_index.astype(jnp.int32)
    npadlen = EPAD - E
    pad_i = jnp.arange(npadlen, dtype=jnp.int32)
    src_pad = jnp.concatenate([ei[0], (pad_i * 131) % N]).reshape(NW, NBLK, BLK)
    dst_pad = jnp.concatenate([ei[1], N + pad_i % (NPAD - N)]).reshape(
        NW, NBLK, BLK)
    src_pad2 = src_pad.reshape(NW, NBLK // 2, 2 * BLK)
    dst_pad2 = dst_pad.reshape(NW, NBLK // 2, 2 * BLK)

    perm = jax.random.permutation(jax.random.key(42), N)
    mask_nodes = perm[:NUM_MASK]
    m = jnp.zeros((N,), f32).at[mask_nodes].set(1.0).reshape(N, 1)

    m_pad = jnp.concatenate([m[:, 0], jnp.zeros((NPAD - N,), f32)])
    degp, fsrc, fdst, fcnt = _deg_sc(dst_pad.reshape(NW, EW),
                                     src_pad.reshape(NW, EW), m_pad)
    dis = _tc_dis(degp)
    t1 = _tc_stage1(dis, x, m, pos_token, W1)

    src_pad1 = src_pad
    dst_pad1 = dst_pad
    part1 = _prop128(t1, src_pad1, dst_pad1)
    t2 = _tc_stage2(part1[0], part1[1], t1, dis, b1.reshape(1, 128),
                    a_enc.reshape(1, 1), W2)

    part2 = _prop64(t2, src_pad2, dst_pad2)
    t3 = _tc_stage3(part2[0], part2[1], t2, dis, b2.reshape(1, 64),
                    a_enc.reshape(1, 1), W_e2d, Wd, m)

    part3 = _prop_masked128(t3, fsrc.reshape(NW, NBLK, BLK),
                            fdst.reshape(NW, NBLK, BLK), fcnt)
    out = _tc_stage4(part3[0], part3[1], t3, dis, bd.reshape(1, 128),
                     a_dec.reshape(1, 1), x, m)
    return out[0, 0]
